# baseline (speedup 1.0000x reference)
import functools

import jax
import jax.numpy as jnp
from jax import lax
from jax.experimental import pallas as pl
from jax.experimental.pallas import tpu as pltpu
from jax.experimental.pallas import tpu_sc as plsc

B, V, C = 16, 50, 512
D, DFF = 256, 1024

G = 8
R = G * V

NC, NS = 2, 16
NW = NC * NS
DH = D // 2
VG = 56
NROWS = (NW // 2) * VG


def _sc_embed(cx_rows, table1):
    mesh = plsc.VectorSubcoreMesh(core_axis_name="c", subcore_axis_name="s")

    @functools.partial(
        pl.kernel, mesh=mesh,
        out_type=jax.ShapeDtypeStruct((2, NROWS, DH), jnp.float32),
        scratch_types=[
            pltpu.VMEM((C, DH), jnp.float32),
            pltpu.VMEM((VG, C), jnp.float32),
            pltpu.VMEM((VG, DH), jnp.float32),
        ],
    )
    def k(cx_hbm, tbl_hbm, out_hbm, tbl_v, cx_v, out_v):
        cid = lax.axis_index("c")
        sid = lax.axis_index("s")
        wid = sid * NC + cid
        dh = wid % 2
        grp = wid // 2
        pltpu.sync_copy(tbl_hbm.at[dh], tbl_v)
        pltpu.sync_copy(cx_hbm.at[pl.ds(grp * VG, VG)], cx_v)
        lane = lax.iota(jnp.int32, 16)

        def visit_body(n, _):
            def chunk_body(cc, accs):
                cx_chunk = cx_v[n, pl.ds(cc * 16, 16)]
                def j_step(j, accs):
                    idxv = lane * 0 + j
                    dn = lax.GatherDimensionNumbers(
                        offset_dims=(), collapsed_slice_dims=(0,),
                        start_index_map=(0,))
                    splat = lax.gather(
                        cx_chunk, idxv[:, None], dn, slice_sizes=(1,),
                        mode=lax.GatherScatterMode.PROMISE_IN_BOUNDS)
                    row = cc * 16 + j
                    accs = tuple(
                        accs[t] + splat * tbl_v[row, pl.ds(t * 16, 16)]
                        for t in range(DH // 16))
                    return accs
                return lax.fori_loop(0, 16, j_step, accs, unroll=True)
            accs = tuple(jnp.zeros((16,), jnp.float32) for _ in range(DH // 16))
            accs = lax.fori_loop(0, C // 16, chunk_body, accs)
            for t in range(DH // 16):
                out_v[n, pl.ds(t * 16, 16)] = accs[t]
            return 0
        lax.fori_loop(0, VG, visit_body, 0)
        pltpu.sync_copy(out_v, out_hbm.at[dh, pl.ds(grp * VG, VG)])

    cx_pad = jnp.zeros((NROWS, C), jnp.float32).at[:B * V].set(cx_rows)
    halves = k(cx_pad, table1.reshape(C, 2, DH).transpose(1, 0, 2))
    return jnp.concatenate([halves[0, :B * V], halves[1, :B * V]], axis=-1)


def _layer_norm(x):
    m = jnp.mean(x, axis=-1, keepdims=True)
    v = jnp.mean((x - m) ** 2, axis=-1, keepdims=True)
    return (x - m) / jnp.sqrt(v + 1e-5)


def _tc_kernel(h_ref, pos_ref, wq_ref, wk_ref, wv_ref,
               wo_ref, w1_ref, w2_ref, wout_ref, out_ref):
    h = h_ref[...]
    tr = jax.lax.broadcasted_iota(jnp.int32, (R, V), 0) % V
    tv = jax.lax.broadcasted_iota(jnp.int32, (R, V), 1)
    tile_op = jnp.where(tr == tv, jnp.float32(1.0), 0.0)
    h = h + jnp.dot(tile_op, pos_ref[...],
                    preferred_element_type=jnp.float32)
    q = jnp.dot(h, wq_ref[...], preferred_element_type=jnp.float32)
    k = jnp.dot(h, wk_ref[...], preferred_element_type=jnp.float32)
    v = jnp.dot(h, wv_ref[...], preferred_element_type=jnp.float32)
    scores = jax.lax.dot_general(
        q, k, (((1,), (1,)), ((), ())),
        preferred_element_type=jnp.float32) * (1.0 / jnp.sqrt(jnp.float32(D)))
    ri = jax.lax.broadcasted_iota(jnp.int32, (R, R), 0) // V
    ci = jax.lax.broadcasted_iota(jnp.int32, (R, R), 1) // V
    scores = jnp.where(ri == ci, scores, -1e30)
    scores = scores - jnp.max(scores, axis=-1, keepdims=True)
    e = jnp.exp(scores)
    attn = e / jnp.sum(e, axis=-1, keepdims=True)
    av = jnp.dot(attn, v, preferred_element_type=jnp.float32)
    h = _layer_norm(h + jnp.dot(av, wo_ref[...],
                                preferred_element_type=jnp.float32))
    ff = jnp.maximum(jnp.dot(h, w1_ref[...],
                             preferred_element_type=jnp.float32), 0.0)
    h = _layer_norm(h + jnp.dot(ff, w2_ref[...],
                                preferred_element_type=jnp.float32))
    pg = jax.lax.broadcasted_iota(jnp.int32, (G, R), 0)
    pr = jax.lax.broadcasted_iota(jnp.int32, (G, R), 1) // V
    pool = jnp.where(pg == pr, jnp.float32(1.0 / V), 0.0)
    pooled = jnp.dot(pool, h, preferred_element_type=jnp.float32)
    out_ref[:, 0, :] = jnp.dot(pooled, wout_ref[...],
                               preferred_element_type=jnp.float32)


def _const_spec(shape):
    return pl.BlockSpec(shape, lambda s: (0,) * len(shape))


@jax.jit
def _run(code_x, table, pos, Wq, Wk, Wv, Wo, W1, W2, Wout):
    cx_rows = code_x.reshape(B * V, C)
    h_raw = _sc_embed(cx_rows, table[1:])
    out = pl.pallas_call(
        _tc_kernel,
        grid=(B // G,),
        in_specs=[
            pl.BlockSpec((R, D), lambda s: (s, 0)),
            _const_spec((V, D)),
            _const_spec((D, D)),
            _const_spec((D, D)),
            _const_spec((D, D)),
            _const_spec((D, D)),
            _const_spec((D, DFF)),
            _const_spec((DFF, D)),
            _const_spec((D, C)),
        ],
        out_specs=pl.BlockSpec((G, 1, C), lambda s: (s, 0, 0)),
        out_shape=jax.ShapeDtypeStruct((B, 1, C), jnp.float32),
    )(h_raw, pos, Wq, Wk, Wv, Wo, W1, W2, Wout)
    return out.reshape(B, C)


def kernel(code_x, divided, neighbors, table, pos, Wq, Wk, Wv, Wo,
           W1, W2, Wout, visit_lens):
    del divided, neighbors, visit_lens
    return _run(code_x, table, pos, Wq, Wk, Wv, Wo, W1, W2, Wout)


# speedup vs baseline: 16.5844x; 16.5844x over previous
import jax
import jax.numpy as jnp
from jax.experimental import pallas as pl
from jax.experimental.pallas import tpu as pltpu

B, V, C = 16, 50, 512
D, DFF = 256, 1024

G = 8
R = G * V


def _layer_norm(x):
    m = jnp.mean(x, axis=-1, keepdims=True)
    v = jnp.mean((x - m) ** 2, axis=-1, keepdims=True)
    return (x - m) / jnp.sqrt(v + 1e-5)


def _fused_kernel(cx_ref, table_ref, pos_ref, wq_ref, wk_ref, wv_ref,
                  wo_hbm, w1_hbm, w2_hbm, wout_hbm, out_ref,
                  wo_v, w1_v, w2_v, wout_v, sem):
    first = pl.program_id(0) == 0

    @pl.when(first)
    def _start_weight_copies():
        pltpu.make_async_copy(wo_hbm, wo_v, sem.at[0]).start()
        pltpu.make_async_copy(w1_hbm, w1_v, sem.at[1]).start()
        pltpu.make_async_copy(w2_hbm, w2_v, sem.at[2]).start()
        pltpu.make_async_copy(wout_hbm, wout_v, sem.at[3]).start()

    cx = cx_ref[...]
    h = jnp.dot(cx, table_ref[pl.ds(1, C), :],
                preferred_element_type=jnp.float32)
    tr = jax.lax.broadcasted_iota(jnp.int32, (R, V), 0) % V
    tv = jax.lax.broadcasted_iota(jnp.int32, (R, V), 1)
    tile_op = jnp.where(tr == tv, jnp.float32(1.0), 0.0)
    h = h + jnp.dot(tile_op, pos_ref[...],
                    preferred_element_type=jnp.float32)
    q = jnp.dot(h, wq_ref[...], preferred_element_type=jnp.float32)
    k = jnp.dot(h, wk_ref[...], preferred_element_type=jnp.float32)
    v = jnp.dot(h, wv_ref[...], preferred_element_type=jnp.float32)
    scores = jax.lax.dot_general(
        q, k, (((1,), (1,)), ((), ())),
        preferred_element_type=jnp.float32) * (1.0 / jnp.sqrt(jnp.float32(D)))
    ri = jax.lax.broadcasted_iota(jnp.int32, (R, R), 0) // V
    ci = jax.lax.broadcasted_iota(jnp.int32, (R, R), 1) // V
    scores = jnp.where(ri == ci, scores, -1e30)
    scores = scores - jnp.max(scores, axis=-1, keepdims=True)
    e = jnp.exp(scores)
    attn = e / jnp.sum(e, axis=-1, keepdims=True)
    av = jnp.dot(attn, v, preferred_element_type=jnp.float32)

    @pl.when(first)
    def _wait_weight_copies():
        pltpu.make_async_copy(wo_hbm, wo_v, sem.at[0]).wait()
        pltpu.make_async_copy(w1_hbm, w1_v, sem.at[1]).wait()
        pltpu.make_async_copy(w2_hbm, w2_v, sem.at[2]).wait()
        pltpu.make_async_copy(wout_hbm, wout_v, sem.at[3]).wait()

    h = _layer_norm(h + jnp.dot(av, wo_v[...],
                                preferred_element_type=jnp.float32))
    ff = jnp.maximum(jnp.dot(h, w1_v[...],
                             preferred_element_type=jnp.float32), 0.0)
    h = _layer_norm(h + jnp.dot(ff, w2_v[...],
                                preferred_element_type=jnp.float32))
    pg = jax.lax.broadcasted_iota(jnp.int32, (G, R), 0)
    pr = jax.lax.broadcasted_iota(jnp.int32, (G, R), 1) // V
    pool = jnp.where(pg == pr, jnp.float32(1.0 / V), 0.0)
    pooled = jnp.dot(pool, h, preferred_element_type=jnp.float32)
    out_ref[:, 0, :] = jnp.dot(pooled, wout_v[...],
                               preferred_element_type=jnp.float32)


def _const_spec(shape):
    return pl.BlockSpec(shape, lambda s: (0,) * len(shape))


@jax.jit
def _run(code_x, table, pos, Wq, Wk, Wv, Wo, W1, W2, Wout):
    cx_rows = code_x.reshape(B * V, C)
    out = pl.pallas_call(
        _fused_kernel,
        grid=(B // G,),
        in_specs=[
            pl.BlockSpec((R, C), lambda s: (s, 0)),
            _const_spec((C + 1, D)),
            _const_spec((V, D)),
            _const_spec((D, D)),
            _const_spec((D, D)),
            _const_spec((D, D)),
            pl.BlockSpec(memory_space=pl.ANY),
            pl.BlockSpec(memory_space=pl.ANY),
            pl.BlockSpec(memory_space=pl.ANY),
            pl.BlockSpec(memory_space=pl.ANY),
        ],
        out_specs=pl.BlockSpec((G, 1, C), lambda s: (s, 0, 0)),
        out_shape=jax.ShapeDtypeStruct((B, 1, C), jnp.float32),
        scratch_shapes=[
            pltpu.VMEM((D, D), jnp.float32),
            pltpu.VMEM((D, DFF), jnp.float32),
            pltpu.VMEM((DFF, D), jnp.float32),
            pltpu.VMEM((D, C), jnp.float32),
            pltpu.SemaphoreType.DMA((4,)),
        ],
    )(cx_rows, table, pos, Wq, Wk, Wv, Wo, W1, W2, Wout)
    return out.reshape(B, C)


def kernel(code_x, divided, neighbors, table, pos, Wq, Wk, Wv, Wo,
           W1, W2, Wout, visit_lens):
    del divided, neighbors, visit_lens
    return _run(code_x, table, pos, Wq, Wk, Wv, Wo, W1, W2, Wout)
